# baseline (speedup 1.0000x reference)
import functools

import jax
import jax.numpy as jnp
from jax import lax
from jax.experimental import pallas as pl
from jax.experimental.pallas import tpu as pltpu
from jax.experimental.pallas import tpu_sc as plsc

N = 10000
E = 160000
E_TOT = E + N
CHUNK = 64
N_TILES = 16
E_PAD = 172032
N_CHUNKS = E_PAD // CHUNK
PER_TILE = N_CHUNKS // N_TILES
PER_TILE_E = E_PAD // N_TILES
NPAD = 10240
STRIPE = NPAD // N_TILES
OSTRIPE = N // N_TILES
DROWS = NPAD // 16
BM = 1000


def _sc_mesh():
    return plsc.VectorSubcoreMesh(core_axis_name="c", subcore_axis_name="s",
                                  num_cores=2, num_subcores=N_TILES)


@functools.cache
def _make_agg(C):
    @functools.partial(
        pl.kernel,
        out_type=jax.ShapeDtypeStruct((2 * N, C), jnp.bfloat16),
        mesh=_sc_mesh(),
        compiler_params=pltpu.CompilerParams(use_tc_tiling_on_sc=False),
        scratch_types=[
            pltpu.VMEM((PER_TILE, CHUNK), jnp.int32),
            pltpu.VMEM((PER_TILE, CHUNK), jnp.int32),
        ] + [pltpu.VMEM((CHUNK, C), jnp.bfloat16)] * 12 + [
            pltpu.VMEM_SHARED((NPAD, C), jnp.bfloat16),
        ] + [pltpu.SemaphoreType.DMA] * 24,
    )
    def agg(src_hbm, dst_hbm, y_hbm, out_hbm, src_v, dst_v, *rest):
        bufs = rest[:12]
        acc = rest[12]
        sg = rest[13:25]
        ss = rest[25:37]
        c = lax.axis_index("c")
        s = lax.axis_index("s")
        pltpu.sync_copy(src_hbm.at[c, pl.ds(s * PER_TILE, PER_TILE)], src_v)
        pltpu.sync_copy(dst_hbm.at[pl.ds(s * PER_TILE, PER_TILE)], dst_v)

        zeros32 = jnp.zeros((32,), jnp.bfloat16)
        per_row = C // 32

        def zero_body(i, carry):
            bufs[0][i // per_row, pl.ds((i % per_row) * 32, 32)] = zeros32
            return carry

        lax.fori_loop(0, CHUNK * per_row, zero_body, 0)
        off = 0
        while off < STRIPE:
            nrows = min(CHUNK, STRIPE - off)
            pltpu.sync_copy(bufs[0].at[pl.ds(0, nrows)],
                            acc.at[pl.ds(s * STRIPE + off, nrows)])
            off += nrows
        plsc.subcore_barrier()

        R = 12
        for t in range(R - 1):
            pltpu.async_copy(y_hbm.at[src_v.at[t]], bufs[t], sg[t])

        def step(g, carry):
            for b in range(R):
                j = g * R + b
                buf = bufs[b]
                pltpu.make_async_copy(y_hbm.at[src_v.at[j]], buf, sg[b]).wait()
                pltpu.async_copy(buf, acc.at[dst_v.at[j]], ss[b], add=True)
                nb = (b + R - 1) % R

                @pl.when(j + R - 1 < PER_TILE)
                def _():
                    @pl.when(j >= 1)
                    def _():
                        pltpu.make_async_copy(
                            bufs[nb], acc.at[dst_v.at[j]], ss[nb]).wait()
                    pltpu.async_copy(y_hbm.at[src_v.at[j + R - 1]],
                                     bufs[nb], sg[nb])
            return carry

        lax.fori_loop(0, PER_TILE // R, step, 0)
        for t in range(PER_TILE - R, PER_TILE):
            pltpu.make_async_copy(bufs[t % R], acc.at[dst_v.at[0]],
                                  ss[t % R]).wait()
        plsc.subcore_barrier()
        pltpu.sync_copy(acc.at[pl.ds(s * OSTRIPE, OSTRIPE)],
                        out_hbm.at[pl.ds(c * N + s * OSTRIPE, OSTRIPE)])

    return agg


@functools.cache
def _make_deg():
    @functools.partial(
        pl.kernel,
        out_type=jax.ShapeDtypeStruct((NPAD, 16), jnp.float32),
        mesh=_sc_mesh(),
        compiler_params=pltpu.CompilerParams(use_tc_tiling_on_sc=False),
        scratch_types=[
            pltpu.VMEM((PER_TILE, CHUNK), jnp.int32),
            pltpu.VMEM((CHUNK, 16), jnp.float32),
            pltpu.VMEM((CHUNK, 16), jnp.float32),
            pltpu.VMEM_SHARED((NPAD, 16), jnp.float32),
            pltpu.SemaphoreType.DMA,
        ],
    )
    def deg(dst_hbm, deg_hbm, dst_v, ones_v, zeros_v, acc, sem):
        c = lax.axis_index("c")
        s = lax.axis_index("s")
        pltpu.sync_copy(dst_hbm.at[pl.ds(s * PER_TILE, PER_TILE)], dst_v)
        zeros16 = jnp.zeros((16,), jnp.float32)
        ones16 = jnp.ones((16,), jnp.float32)

        def fill_body(i, carry):
            ones_v[i, :] = ones16
            zeros_v[i, :] = zeros16
            return carry

        lax.fori_loop(0, CHUNK, fill_body, 0)
        off = 0
        while off < STRIPE:
            nrows = min(CHUNK, STRIPE - off)
            pltpu.sync_copy(zeros_v.at[pl.ds(0, nrows)],
                            acc.at[pl.ds(s * STRIPE + off, nrows)])
            off += nrows
        plsc.subcore_barrier()

        W = 16

        def fire(j, carry):
            pltpu.async_copy(ones_v, acc.at[dst_v.at[j]], sem, add=True)

            @pl.when(j >= W)
            def _():
                pltpu.make_async_copy(ones_v, acc.at[dst_v.at[0]], sem).wait()
            return carry

        lax.fori_loop(0, PER_TILE, fire, 0)

        def drain(j, carry):
            pltpu.make_async_copy(ones_v, acc.at[dst_v.at[0]], sem).wait()
            return carry

        lax.fori_loop(0, W, drain, 0)
        plsc.subcore_barrier()

        @pl.when(c == 0)
        def _():
            pltpu.sync_copy(acc.at[pl.ds(s * STRIPE, STRIPE)],
                            deg_hbm.at[pl.ds(s * STRIPE, STRIPE)])

    return deg


def _mm_first(x, W, deg2d):
    k, n = W.shape
    ch = n // 2
    Ws = W.reshape(k, 2, ch).transpose(1, 0, 2).astype(jnp.bfloat16)

    def body(x_ref, w_ref, deg_ref, o_ref):
        dinv = lax.rsqrt(deg_ref[...])
        o_ref[...] = (jnp.dot(x_ref[...], w_ref[0],
                              preferred_element_type=jnp.float32)
                      * dinv).astype(jnp.bfloat16)

    return pl.pallas_call(
        body,
        grid=(N // BM, 2),
        in_specs=[
            pl.BlockSpec((BM, x.shape[1]), lambda i, j: (i, 0)),
            pl.BlockSpec((1, k, ch), lambda i, j: (j, 0, 0)),
            pl.BlockSpec((BM, 1), lambda i, j: (i, 0)),
        ],
        out_specs=pl.BlockSpec((BM, ch), lambda i, j: (j * (N // BM) + i, 0)),
        out_shape=jax.ShapeDtypeStruct((2 * N, ch), jnp.bfloat16),
    )(x.astype(jnp.bfloat16), Ws, deg2d)


def _mm_mid(agg_lo, agg_hi, deg2d, b2d, W):
    k, n = W.shape
    ch = n // 2
    half = agg_lo.shape[1]
    Ws = W.reshape(k, 2, ch).transpose(1, 0, 2).astype(jnp.bfloat16)

    def body(lo_ref, hi_ref, deg_ref, b_ref, w_ref, o_ref):
        dinv = lax.rsqrt(deg_ref[...])
        h = jnp.concatenate([lo_ref[...], hi_ref[...]],
                            axis=1).astype(jnp.float32) * dinv
        h = jnp.maximum(h + b_ref[...], 0.0)
        o_ref[...] = (jnp.dot(h.astype(jnp.bfloat16), w_ref[0],
                              preferred_element_type=jnp.float32)
                      * dinv).astype(jnp.bfloat16)

    return pl.pallas_call(
        body,
        grid=(N // BM, 2),
        in_specs=[
            pl.BlockSpec((BM, half), lambda i, j: (i, 0)),
            pl.BlockSpec((BM, half), lambda i, j: (i, 0)),
            pl.BlockSpec((BM, 1), lambda i, j: (i, 0)),
            pl.BlockSpec((1, 2 * half), lambda i, j: (0, 0)),
            pl.BlockSpec((1, k, ch), lambda i, j: (j, 0, 0)),
        ],
        out_specs=pl.BlockSpec((BM, ch), lambda i, j: (j * (N // BM) + i, 0)),
        out_shape=jax.ShapeDtypeStruct((2 * N, ch), jnp.bfloat16),
    )(agg_lo, agg_hi, deg2d, b2d, Ws)


def _epilogue(agg_lo, agg_hi, deg2d, b2d):
    half = agg_lo.shape[1]

    def body(lo_ref, hi_ref, deg_ref, b_ref, o_ref):
        dinv = lax.rsqrt(deg_ref[...])
        h = jnp.concatenate([lo_ref[...], hi_ref[...]],
                            axis=1).astype(jnp.float32) * dinv
        h = jnp.maximum(h + b_ref[...], 0.0)
        m = jnp.max(h, axis=1, keepdims=True)
        lse = jnp.log(jnp.sum(jnp.exp(h - m), axis=1, keepdims=True)) + m
        o_ref[...] = h - lse

    return pl.pallas_call(
        body,
        grid=(N // BM,),
        in_specs=[
            pl.BlockSpec((BM, half), lambda i: (i, 0)),
            pl.BlockSpec((BM, half), lambda i: (i, 0)),
            pl.BlockSpec((BM, 1), lambda i: (i, 0)),
            pl.BlockSpec((1, 2 * half), lambda i: (0, 0)),
        ],
        out_specs=pl.BlockSpec((BM, 2 * half), lambda i: (i, 0)),
        out_shape=jax.ShapeDtypeStruct((N, 2 * half), jnp.float32),
    )(agg_lo, agg_hi, deg2d, b2d)


def kernel(x, edge_index, W1, b1, W2, b2, W3, b3):
    ei = edge_index.astype(jnp.int32)
    loop = jnp.arange(N, dtype=jnp.int32)
    pad = E_PAD - E_TOT
    src = jnp.concatenate([ei[0], loop, jnp.zeros((pad,), jnp.int32)])
    dst = jnp.concatenate([ei[1], loop, jnp.full((pad,), N + 8, jnp.int32)])
    src2 = jnp.stack([src, src + N]).reshape(2, N_CHUNKS, CHUNK)
    dst2 = dst.reshape(N_CHUNKS, CHUNK)

    agg128 = _make_agg(128)
    agg64 = _make_agg(64)
    deg2d = _make_deg()(dst2)[:N, :1]

    y1 = _mm_first(x, W1, deg2d)
    a1 = agg128(src2, dst2, y1)
    y2 = _mm_mid(a1[:N], a1[N:], deg2d, b1.reshape(1, -1), W2)
    a2 = agg128(src2, dst2, y2)
    y3 = _mm_mid(a2[:N], a2[N:], deg2d, b2.reshape(1, -1), W3)
    a3 = agg64(src2, dst2, y3)
    return _epilogue(a3[:N], a3[N:], deg2d, b3.reshape(1, -1))


# speedup vs baseline: 1.0200x; 1.0200x over previous
import functools

import jax
import jax.numpy as jnp
from jax import lax
from jax.experimental import pallas as pl
from jax.experimental.pallas import tpu as pltpu
from jax.experimental.pallas import tpu_sc as plsc

N = 10000
E = 160000
E_TOT = E + N
CHUNK = 64
N_TILES = 16
E_PAD = 172032
N_CHUNKS = E_PAD // CHUNK
PER_TILE = N_CHUNKS // N_TILES
PER_TILE_E = E_PAD // N_TILES
NPAD = 10240
STRIPE = NPAD // N_TILES
OSTRIPE = N // N_TILES
DROWS = NPAD // 16
BM = 1000


def _sc_mesh():
    return plsc.VectorSubcoreMesh(core_axis_name="c", subcore_axis_name="s",
                                  num_cores=2, num_subcores=N_TILES)


@functools.cache
def _make_agg(C):
    @functools.partial(
        pl.kernel,
        out_type=jax.ShapeDtypeStruct((2 * N, C), jnp.bfloat16),
        mesh=_sc_mesh(),
        compiler_params=pltpu.CompilerParams(use_tc_tiling_on_sc=False),
        scratch_types=[
            pltpu.VMEM((PER_TILE, CHUNK), jnp.int32),
            pltpu.VMEM((PER_TILE, CHUNK), jnp.int32),
        ] + [pltpu.VMEM((CHUNK, C), jnp.bfloat16)] * 6 + [
            pltpu.VMEM_SHARED((NPAD, C), jnp.bfloat16),
        ] + [pltpu.SemaphoreType.DMA] * 12,
    )
    def agg(src_hbm, dst_hbm, y_hbm, out_hbm, src_v, dst_v, *rest):
        bufs = rest[:6]
        acc = rest[6]
        sg = rest[7:13]
        ss = rest[13:19]
        c = lax.axis_index("c")
        s = lax.axis_index("s")
        pltpu.sync_copy(src_hbm.at[c, pl.ds(s * PER_TILE, PER_TILE)], src_v)
        pltpu.sync_copy(dst_hbm.at[pl.ds(s * PER_TILE, PER_TILE)], dst_v)

        zeros32 = jnp.zeros((32,), jnp.bfloat16)
        per_row = C // 32

        def zero_body(i, carry):
            bufs[0][i // per_row, pl.ds((i % per_row) * 32, 32)] = zeros32
            return carry

        lax.fori_loop(0, CHUNK * per_row, zero_body, 0)
        off = 0
        while off < STRIPE:
            nrows = min(CHUNK, STRIPE - off)
            pltpu.sync_copy(bufs[0].at[pl.ds(0, nrows)],
                            acc.at[pl.ds(s * STRIPE + off, nrows)])
            off += nrows
        plsc.subcore_barrier()

        R = 6
        for t in range(R - 1):
            pltpu.async_copy(y_hbm.at[src_v.at[t]], bufs[t], sg[t])

        def step(g, carry):
            for b in range(R):
                j = g * R + b
                buf = bufs[b]
                pltpu.make_async_copy(y_hbm.at[src_v.at[j]], buf, sg[b]).wait()
                pltpu.async_copy(buf, acc.at[dst_v.at[j]], ss[b], add=True)
                nb = (b + R - 1) % R

                @pl.when(j + R - 1 < PER_TILE)
                def _():
                    @pl.when(j >= 1)
                    def _():
                        pltpu.make_async_copy(
                            bufs[nb], acc.at[dst_v.at[j]], ss[nb]).wait()
                    pltpu.async_copy(y_hbm.at[src_v.at[j + R - 1]],
                                     bufs[nb], sg[nb])
            return carry

        lax.fori_loop(0, PER_TILE // R, step, 0)
        for t in range(PER_TILE - R, PER_TILE):
            pltpu.make_async_copy(bufs[t % R], acc.at[dst_v.at[0]],
                                  ss[t % R]).wait()
        plsc.subcore_barrier()
        pltpu.sync_copy(acc.at[pl.ds(s * OSTRIPE, OSTRIPE)],
                        out_hbm.at[pl.ds(c * N + s * OSTRIPE, OSTRIPE)])

    return agg


@functools.cache
def _make_deg():
    @functools.partial(
        pl.kernel,
        out_type=jax.ShapeDtypeStruct((NPAD, 16), jnp.float32),
        mesh=_sc_mesh(),
        compiler_params=pltpu.CompilerParams(use_tc_tiling_on_sc=False),
        scratch_types=[
            pltpu.VMEM((PER_TILE, CHUNK), jnp.int32),
            pltpu.VMEM((CHUNK, 16), jnp.float32),
            pltpu.VMEM((CHUNK, 16), jnp.float32),
            pltpu.VMEM_SHARED((NPAD, 16), jnp.float32),
            pltpu.SemaphoreType.DMA,
        ],
    )
    def deg(dst_hbm, deg_hbm, dst_v, ones_v, zeros_v, acc, sem):
        c = lax.axis_index("c")
        s = lax.axis_index("s")
        pltpu.sync_copy(dst_hbm.at[pl.ds(s * PER_TILE, PER_TILE)], dst_v)
        zeros16 = jnp.zeros((16,), jnp.float32)
        ones16 = jnp.ones((16,), jnp.float32)

        def fill_body(i, carry):
            ones_v[i, :] = ones16
            zeros_v[i, :] = zeros16
            return carry

        lax.fori_loop(0, CHUNK, fill_body, 0)
        off = 0
        while off < STRIPE:
            nrows = min(CHUNK, STRIPE - off)
            pltpu.sync_copy(zeros_v.at[pl.ds(0, nrows)],
                            acc.at[pl.ds(s * STRIPE + off, nrows)])
            off += nrows
        plsc.subcore_barrier()

        W = 16

        def fire(j, carry):
            pltpu.async_copy(ones_v, acc.at[dst_v.at[j]], sem, add=True)

            @pl.when(j >= W)
            def _():
                pltpu.make_async_copy(ones_v, acc.at[dst_v.at[0]], sem).wait()
            return carry

        lax.fori_loop(0, PER_TILE, fire, 0)

        def drain(j, carry):
            pltpu.make_async_copy(ones_v, acc.at[dst_v.at[0]], sem).wait()
            return carry

        lax.fori_loop(0, W, drain, 0)
        plsc.subcore_barrier()

        @pl.when(c == 0)
        def _():
            pltpu.sync_copy(acc.at[pl.ds(s * STRIPE, STRIPE)],
                            deg_hbm.at[pl.ds(s * STRIPE, STRIPE)])

    return deg


def _mm_first(x, W, deg2d):
    k, n = W.shape
    ch = n // 2
    Ws = W.reshape(k, 2, ch).transpose(1, 0, 2).astype(jnp.bfloat16)

    def body(x_ref, w_ref, deg_ref, o_ref):
        dinv = lax.rsqrt(deg_ref[...])
        o_ref[...] = (jnp.dot(x_ref[...], w_ref[0],
                              preferred_element_type=jnp.float32)
                      * dinv).astype(jnp.bfloat16)

    return pl.pallas_call(
        body,
        grid=(N // BM, 2),
        in_specs=[
            pl.BlockSpec((BM, x.shape[1]), lambda i, j: (i, 0)),
            pl.BlockSpec((1, k, ch), lambda i, j: (j, 0, 0)),
            pl.BlockSpec((BM, 1), lambda i, j: (i, 0)),
        ],
        out_specs=pl.BlockSpec((BM, ch), lambda i, j: (j * (N // BM) + i, 0)),
        out_shape=jax.ShapeDtypeStruct((2 * N, ch), jnp.bfloat16),
    )(x.astype(jnp.bfloat16), Ws, deg2d)


def _mm_mid(agg_lo, agg_hi, deg2d, b2d, W):
    k, n = W.shape
    ch = n // 2
    half = agg_lo.shape[1]
    Ws = W.reshape(k, 2, ch).transpose(1, 0, 2).astype(jnp.bfloat16)

    def body(lo_ref, hi_ref, deg_ref, b_ref, w_ref, o_ref):
        dinv = lax.rsqrt(deg_ref[...])
        h = jnp.concatenate([lo_ref[...], hi_ref[...]],
                            axis=1).astype(jnp.float32) * dinv
        h = jnp.maximum(h + b_ref[...], 0.0)
        o_ref[...] = (jnp.dot(h.astype(jnp.bfloat16), w_ref[0],
                              preferred_element_type=jnp.float32)
                      * dinv).astype(jnp.bfloat16)

    return pl.pallas_call(
        body,
        grid=(N // BM, 2),
        in_specs=[
            pl.BlockSpec((BM, half), lambda i, j: (i, 0)),
            pl.BlockSpec((BM, half), lambda i, j: (i, 0)),
            pl.BlockSpec((BM, 1), lambda i, j: (i, 0)),
            pl.BlockSpec((1, 2 * half), lambda i, j: (0, 0)),
            pl.BlockSpec((1, k, ch), lambda i, j: (j, 0, 0)),
        ],
        out_specs=pl.BlockSpec((BM, ch), lambda i, j: (j * (N // BM) + i, 0)),
        out_shape=jax.ShapeDtypeStruct((2 * N, ch), jnp.bfloat16),
    )(agg_lo, agg_hi, deg2d, b2d, Ws)


def _epilogue(agg_lo, agg_hi, deg2d, b2d):
    half = agg_lo.shape[1]

    def body(lo_ref, hi_ref, deg_ref, b_ref, o_ref):
        dinv = lax.rsqrt(deg_ref[...])
        h = jnp.concatenate([lo_ref[...], hi_ref[...]],
                            axis=1).astype(jnp.float32) * dinv
        h = jnp.maximum(h + b_ref[...], 0.0)
        m = jnp.max(h, axis=1, keepdims=True)
        lse = jnp.log(jnp.sum(jnp.exp(h - m), axis=1, keepdims=True)) + m
        o_ref[...] = h - lse

    return pl.pallas_call(
        body,
        grid=(N // BM,),
        in_specs=[
            pl.BlockSpec((BM, half), lambda i: (i, 0)),
            pl.BlockSpec((BM, half), lambda i: (i, 0)),
            pl.BlockSpec((BM, 1), lambda i: (i, 0)),
            pl.BlockSpec((1, 2 * half), lambda i: (0, 0)),
        ],
        out_specs=pl.BlockSpec((BM, 2 * half), lambda i: (i, 0)),
        out_shape=jax.ShapeDtypeStruct((N, 2 * half), jnp.float32),
    )(agg_lo, agg_hi, deg2d, b2d)


def kernel(x, edge_index, W1, b1, W2, b2, W3, b3):
    ei = edge_index.astype(jnp.int32)
    loop = jnp.arange(N, dtype=jnp.int32)
    pad = E_PAD - E_TOT
    src = jnp.concatenate([ei[0], loop, jnp.zeros((pad,), jnp.int32)])
    dst = jnp.concatenate([ei[1], loop, jnp.full((pad,), N + 8, jnp.int32)])
    src2 = jnp.stack([src, src + N]).reshape(2, N_CHUNKS, CHUNK)
    dst2 = dst.reshape(N_CHUNKS, CHUNK)

    agg128 = _make_agg(128)
    agg64 = _make_agg(64)
    deg2d = _make_deg()(dst2)[:N, :1]

    y1 = _mm_first(x, W1, deg2d)
    a1 = agg128(src2, dst2, y1)
    y2 = _mm_mid(a1[:N], a1[N:], deg2d, b1.reshape(1, -1), W2)
    a2 = agg128(src2, dst2, y2)
    y3 = _mm_mid(a2[:N], a2[N:], deg2d, b2.reshape(1, -1), W3)
    a3 = agg64(src2, dst2, y3)
    return _epilogue(a3[:N], a3[N:], deg2d, b3.reshape(1, -1))


# speedup vs baseline: 1.0346x; 1.0144x over previous
import functools

import jax
import jax.numpy as jnp
from jax import lax
from jax.experimental import pallas as pl
from jax.experimental.pallas import tpu as pltpu
from jax.experimental.pallas import tpu_sc as plsc

N = 10000
E = 160000
E_TOT = E + N
CHUNK = 64
N_TILES = 16
E_PAD = 172032
N_CHUNKS = E_PAD // CHUNK
PER_TILE = N_CHUNKS // N_TILES
PER_TILE_E = E_PAD // N_TILES
NPAD = 10240
STRIPE = NPAD // N_TILES
OSTRIPE = N // N_TILES
DROWS = NPAD // 16
BM = 1000


def _sc_mesh():
    return plsc.VectorSubcoreMesh(core_axis_name="c", subcore_axis_name="s",
                                  num_cores=2, num_subcores=N_TILES)


@functools.cache
def _make_agg(C):
    @functools.partial(
        pl.kernel,
        out_type=(jax.ShapeDtypeStruct((N, C), jnp.bfloat16),
                  jax.ShapeDtypeStruct((N, C), jnp.bfloat16)),
        mesh=_sc_mesh(),
        compiler_params=pltpu.CompilerParams(use_tc_tiling_on_sc=False),
        scratch_types=[
            pltpu.VMEM((PER_TILE, CHUNK), jnp.int32),
            pltpu.VMEM((PER_TILE, CHUNK), jnp.int32),
        ] + [pltpu.VMEM((CHUNK, C), jnp.bfloat16)] * 6 + [
            pltpu.VMEM_SHARED((NPAD, C), jnp.bfloat16),
        ] + [pltpu.SemaphoreType.DMA] * 12,
    )
    def agg(src_hbm, dst_hbm, y_hbm, out_lo, out_hi, src_v, dst_v, *rest):
        bufs = rest[:6]
        acc = rest[6]
        sg = rest[7:13]
        ss = rest[13:19]
        c = lax.axis_index("c")
        s = lax.axis_index("s")
        pltpu.sync_copy(src_hbm.at[c, pl.ds(s * PER_TILE, PER_TILE)], src_v)
        pltpu.sync_copy(dst_hbm.at[pl.ds(s * PER_TILE, PER_TILE)], dst_v)

        zeros32 = jnp.zeros((32,), jnp.bfloat16)
        per_row = C // 32

        def zero_body(i, carry):
            bufs[0][i // per_row, pl.ds((i % per_row) * 32, 32)] = zeros32
            return carry

        lax.fori_loop(0, CHUNK * per_row, zero_body, 0)
        off = 0
        while off < STRIPE:
            nrows = min(CHUNK, STRIPE - off)
            pltpu.sync_copy(bufs[0].at[pl.ds(0, nrows)],
                            acc.at[pl.ds(s * STRIPE + off, nrows)])
            off += nrows
        plsc.subcore_barrier()

        R = 6
        for t in range(R - 1):
            pltpu.async_copy(y_hbm.at[src_v.at[t]], bufs[t], sg[t])

        def step(g, carry):
            for b in range(R):
                j = g * R + b
                buf = bufs[b]
                pltpu.make_async_copy(y_hbm.at[src_v.at[j]], buf, sg[b]).wait()
                pltpu.async_copy(buf, acc.at[dst_v.at[j]], ss[b], add=True)
                nb = (b + R - 1) % R

                @pl.when(j + R - 1 < PER_TILE)
                def _():
                    @pl.when(j >= 1)
                    def _():
                        pltpu.make_async_copy(
                            bufs[nb], acc.at[dst_v.at[j]], ss[nb]).wait()
                    pltpu.async_copy(y_hbm.at[src_v.at[j + R - 1]],
                                     bufs[nb], sg[nb])
            return carry

        lax.fori_loop(0, PER_TILE // R, step, 0)
        for t in range(PER_TILE - R, PER_TILE):
            pltpu.make_async_copy(bufs[t % R], acc.at[dst_v.at[0]],
                                  ss[t % R]).wait()
        plsc.subcore_barrier()

        @pl.when(c == 0)
        def _():
            pltpu.sync_copy(acc.at[pl.ds(s * OSTRIPE, OSTRIPE)],
                            out_lo.at[pl.ds(s * OSTRIPE, OSTRIPE)])

        @pl.when(c == 1)
        def _():
            pltpu.sync_copy(acc.at[pl.ds(s * OSTRIPE, OSTRIPE)],
                            out_hi.at[pl.ds(s * OSTRIPE, OSTRIPE)])

    return agg


@functools.cache
def _make_deg():
    @functools.partial(
        pl.kernel,
        out_type=jax.ShapeDtypeStruct((NPAD, 16), jnp.float32),
        mesh=_sc_mesh(),
        compiler_params=pltpu.CompilerParams(use_tc_tiling_on_sc=False),
        scratch_types=[
            pltpu.VMEM((PER_TILE, CHUNK), jnp.int32),
            pltpu.VMEM((CHUNK, 16), jnp.float32),
            pltpu.VMEM((CHUNK, 16), jnp.float32),
            pltpu.VMEM_SHARED((NPAD, 16), jnp.float32),
            pltpu.SemaphoreType.DMA,
        ],
    )
    def deg(dst_hbm, deg_hbm, dst_v, ones_v, zeros_v, acc, sem):
        c = lax.axis_index("c")
        s = lax.axis_index("s")
        pltpu.sync_copy(dst_hbm.at[pl.ds(s * PER_TILE, PER_TILE)], dst_v)
        zeros16 = jnp.zeros((16,), jnp.float32)
        ones16 = jnp.ones((16,), jnp.float32)

        def fill_body(i, carry):
            ones_v[i, :] = ones16
            zeros_v[i, :] = zeros16
            return carry

        lax.fori_loop(0, CHUNK, fill_body, 0)
        off = 0
        while off < STRIPE:
            nrows = min(CHUNK, STRIPE - off)
            pltpu.sync_copy(zeros_v.at[pl.ds(0, nrows)],
                            acc.at[pl.ds(s * STRIPE + off, nrows)])
            off += nrows
        plsc.subcore_barrier()

        W = 16

        def fire(j, carry):
            pltpu.async_copy(ones_v, acc.at[dst_v.at[j]], sem, add=True)

            @pl.when(j >= W)
            def _():
                pltpu.make_async_copy(ones_v, acc.at[dst_v.at[0]], sem).wait()
            return carry

        lax.fori_loop(0, PER_TILE, fire, 0)

        def drain(j, carry):
            pltpu.make_async_copy(ones_v, acc.at[dst_v.at[0]], sem).wait()
            return carry

        lax.fori_loop(0, W, drain, 0)
        plsc.subcore_barrier()

        @pl.when(c == 0)
        def _():
            pltpu.sync_copy(acc.at[pl.ds(s * STRIPE, STRIPE)],
                            deg_hbm.at[pl.ds(s * STRIPE, STRIPE)])

    return deg


def _mm_first(x, W, deg2d):
    k, n = W.shape
    ch = n // 2
    Ws = W.reshape(k, 2, ch).transpose(1, 0, 2).astype(jnp.bfloat16)

    def body(x_ref, w_ref, deg_ref, o_ref):
        dinv = lax.rsqrt(deg_ref[...])
        o_ref[...] = (jnp.dot(x_ref[...], w_ref[0],
                              preferred_element_type=jnp.float32)
                      * dinv).astype(jnp.bfloat16)

    return pl.pallas_call(
        body,
        grid=(N // BM, 2),
        in_specs=[
            pl.BlockSpec((BM, x.shape[1]), lambda i, j: (i, 0)),
            pl.BlockSpec((1, k, ch), lambda i, j: (j, 0, 0)),
            pl.BlockSpec((BM, 1), lambda i, j: (i, 0)),
        ],
        out_specs=pl.BlockSpec((BM, ch), lambda i, j: (j * (N // BM) + i, 0)),
        out_shape=jax.ShapeDtypeStruct((2 * N, ch), jnp.bfloat16),
    )(x.astype(jnp.bfloat16), Ws, deg2d)


def _mm_mid(agg_lo, agg_hi, deg2d, b2d, W):
    k, n = W.shape
    ch = n // 2
    half = agg_lo.shape[1]
    Ws = W.reshape(k, 2, ch).transpose(1, 0, 2).astype(jnp.bfloat16)

    def body(lo_ref, hi_ref, deg_ref, b_ref, w_ref, o_ref):
        dinv = lax.rsqrt(deg_ref[...])
        h = jnp.concatenate([lo_ref[...], hi_ref[...]],
                            axis=1).astype(jnp.float32) * dinv
        h = jnp.maximum(h + b_ref[...], 0.0)
        o_ref[...] = (jnp.dot(h.astype(jnp.bfloat16), w_ref[0],
                              preferred_element_type=jnp.float32)
                      * dinv).astype(jnp.bfloat16)

    return pl.pallas_call(
        body,
        grid=(N // BM, 2),
        in_specs=[
            pl.BlockSpec((BM, half), lambda i, j: (i, 0)),
            pl.BlockSpec((BM, half), lambda i, j: (i, 0)),
            pl.BlockSpec((BM, 1), lambda i, j: (i, 0)),
            pl.BlockSpec((1, 2 * half), lambda i, j: (0, 0)),
            pl.BlockSpec((1, k, ch), lambda i, j: (j, 0, 0)),
        ],
        out_specs=pl.BlockSpec((BM, ch), lambda i, j: (j * (N // BM) + i, 0)),
        out_shape=jax.ShapeDtypeStruct((2 * N, ch), jnp.bfloat16),
    )(agg_lo, agg_hi, deg2d, b2d, Ws)


def _epilogue(agg_lo, agg_hi, deg2d, b2d):
    half = agg_lo.shape[1]

    def body(lo_ref, hi_ref, deg_ref, b_ref, o_ref):
        dinv = lax.rsqrt(deg_ref[...])
        h = jnp.concatenate([lo_ref[...], hi_ref[...]],
                            axis=1).astype(jnp.float32) * dinv
        h = jnp.maximum(h + b_ref[...], 0.0)
        m = jnp.max(h, axis=1, keepdims=True)
        lse = jnp.log(jnp.sum(jnp.exp(h - m), axis=1, keepdims=True)) + m
        o_ref[...] = h - lse

    return pl.pallas_call(
        body,
        grid=(N // BM,),
        in_specs=[
            pl.BlockSpec((BM, half), lambda i: (i, 0)),
            pl.BlockSpec((BM, half), lambda i: (i, 0)),
            pl.BlockSpec((BM, 1), lambda i: (i, 0)),
            pl.BlockSpec((1, 2 * half), lambda i: (0, 0)),
        ],
        out_specs=pl.BlockSpec((BM, 2 * half), lambda i: (i, 0)),
        out_shape=jax.ShapeDtypeStruct((N, 2 * half), jnp.float32),
    )(agg_lo, agg_hi, deg2d, b2d)


def kernel(x, edge_index, W1, b1, W2, b2, W3, b3):
    ei = edge_index.astype(jnp.int32)
    loop = jnp.arange(N, dtype=jnp.int32)
    pad = E_PAD - E_TOT
    src = jnp.concatenate([ei[0], loop, jnp.zeros((pad,), jnp.int32)])
    dst = jnp.concatenate([ei[1], loop, jnp.full((pad,), N + 8, jnp.int32)])
    src2 = jnp.stack([src, src + N]).reshape(2, N_CHUNKS, CHUNK)
    dst2 = dst.reshape(N_CHUNKS, CHUNK)

    agg128 = _make_agg(128)
    agg64 = _make_agg(64)
    deg2d = _make_deg()(dst2)[:N, :1]

    y1 = _mm_first(x, W1, deg2d)
    a1lo, a1hi = agg128(src2, dst2, y1)
    y2 = _mm_mid(a1lo, a1hi, deg2d, b1.reshape(1, -1), W2)
    a2lo, a2hi = agg128(src2, dst2, y2)
    y3 = _mm_mid(a2lo, a2hi, deg2d, b2.reshape(1, -1), W3)
    a3lo, a3hi = agg64(src2, dst2, y3)
    return _epilogue(a3lo, a3hi, deg2d, b3.reshape(1, -1))
